# Initial kernel scaffold; baseline (speedup 1.0000x reference)
import jax
import jax.numpy as jnp
from jax.experimental import pallas as pl


def kernel(z, cluster_centers):
    raise NotImplementedError("write your pallas kernel here")


# speedup vs baseline: 1.4412x; 1.4412x over previous
import functools

import jax
import jax.numpy as jnp
from jax.experimental import pallas as pl
from jax.experimental.pallas import tpu as pltpu

_B = 262144
_D = 128
_K = 10
_BB = 8192
_NB = _B // _BB


def _q_kernel(z_ref, c_ref, q_ref, part_ref):
    zb = z_ref[...]
    c = c_ref[...]
    zc = jax.lax.dot_general(
        c, zb, (((1,), (1,)), ((), ())),
        preferred_element_type=jnp.float32)
    zsq = zb * zb
    ones = jnp.ones((1, _D), dtype=jnp.float32)
    zn = jax.lax.dot_general(
        ones, zsq, (((1,), (1,)), ((), ())),
        preferred_element_type=jnp.float32)
    cn = jnp.sum(c * c, axis=1, keepdims=True)
    sq = zn + (cn - 2.0 * zc)
    qu = 1.0 / (1.0 + sq)
    rs = jnp.sum(qu, axis=0, keepdims=True)
    qn = qu * (1.0 / rs)
    acc = qn[:, 0:128]
    for k in range(1, _BB // 128):
        acc = acc + qn[:, k * 128:(k + 1) * 128]
    part_ref[...] = acc
    q_ref[...] = qn.T


def _p_kernel(q_ref, part_ref, p_ref):
    qb = q_ref[...]
    qt = qb.T
    parts = part_ref[...]
    s = parts[:, 0:128]
    for k in range(1, _NB):
        s = s + parts[:, k * 128:(k + 1) * 128]
    s = jnp.sum(s, axis=1, keepdims=True)
    w = (qt * qt) * (1.0 / s)
    rs = jnp.sum(w, axis=0, keepdims=True)
    pt = w * (1.0 / rs)
    p_ref[...] = pt.T


@functools.partial(jax.jit, static_argnames=("interpret",))
def kernel(z, cluster_centers, interpret=False):
    q, parts = pl.pallas_call(
        _q_kernel,
        grid=(_NB,),
        in_specs=[
            pl.BlockSpec((_BB, _D), lambda i: (i, 0)),
            pl.BlockSpec((_K, _D), lambda i: (0, 0)),
        ],
        out_specs=[
            pl.BlockSpec((_BB, _K), lambda i: (i, 0)),
            pl.BlockSpec((_K, 128), lambda i: (0, i)),
        ],
        out_shape=[
            jax.ShapeDtypeStruct((_B, _K), jnp.float32),
            jax.ShapeDtypeStruct((_K, _NB * 128), jnp.float32),
        ],
        compiler_params=pltpu.CompilerParams(
            dimension_semantics=("parallel",),
        ),
        interpret=interpret,
    )(z, cluster_centers)

    p = pl.pallas_call(
        _p_kernel,
        grid=(_NB,),
        in_specs=[
            pl.BlockSpec((_BB, _K), lambda i: (i, 0)),
            pl.BlockSpec((_K, _NB * 128), lambda i: (0, 0)),
        ],
        out_specs=pl.BlockSpec((_BB, _K), lambda i: (i, 0)),
        out_shape=jax.ShapeDtypeStruct((_B, _K), jnp.float32),
        compiler_params=pltpu.CompilerParams(
            dimension_semantics=("parallel",),
        ),
        interpret=interpret,
    )(q, parts)
    return (q, p)


# speedup vs baseline: 1.4647x; 1.0163x over previous
import functools

import jax
import jax.numpy as jnp
from jax.experimental import pallas as pl
from jax.experimental.pallas import tpu as pltpu

_B = 262144
_D = 128
_K = 10
_BB = 4096
_NB = _B // _BB


def _kernel(z_ref, c_ref, q_ref, p_ref, qt_ref, acc_ref):
    c = pl.program_id(0)
    i = pl.program_id(1)
    off = pl.multiple_of(i * _BB, _BB)

    @pl.when(c == 0)
    def _phase_a():
        zb = z_ref[...]
        cc = c_ref[...]
        zc = jax.lax.dot_general(
            cc, zb, (((1,), (1,)), ((), ())),
            preferred_element_type=jnp.float32)
        zsq = zb * zb
        ones = jnp.ones((1, _D), dtype=jnp.float32)
        zn = jax.lax.dot_general(
            ones, zsq, (((1,), (1,)), ((), ())),
            preferred_element_type=jnp.float32)
        cn = jnp.sum(cc * cc, axis=1, keepdims=True)
        sq = zn + (cn - 2.0 * zc)
        qu = 1.0 / (1.0 + sq)
        rs = jnp.sum(qu, axis=0, keepdims=True)
        qn = qu * (1.0 / rs)
        qt_ref[:, pl.ds(off, _BB)] = qn
        part = qn[:, 0:128]
        for k in range(1, _BB // 128):
            part = part + qn[:, k * 128:(k + 1) * 128]

        @pl.when(i == 0)
        def _():
            acc_ref[...] = part

        @pl.when(i > 0)
        def _():
            acc_ref[...] = acc_ref[...] + part

        q_ref[...] = qn.T

    @pl.when(c == 1)
    def _phase_b():
        qt = qt_ref[:, pl.ds(off, _BB)]
        s = jnp.sum(acc_ref[...], axis=1, keepdims=True)
        w = (qt * qt) * (1.0 / s)
        rs = jnp.sum(w, axis=0, keepdims=True)
        pt = w * (1.0 / rs)
        p_ref[...] = pt.T


@functools.partial(jax.jit, static_argnames=("interpret",))
def kernel(z, cluster_centers, interpret=False):
    q, p = pl.pallas_call(
        _kernel,
        grid=(2, _NB),
        in_specs=[
            pl.BlockSpec((_BB, _D), lambda c, i: (jnp.where(c == 0, i, _NB - 1), 0)),
            pl.BlockSpec((_K, _D), lambda c, i: (0, 0)),
        ],
        out_specs=[
            pl.BlockSpec((_BB, _K), lambda c, i: (jnp.where(c == 0, i, _NB - 1), 0)),
            pl.BlockSpec((_BB, _K), lambda c, i: (jnp.where(c == 0, 0, i), 0)),
        ],
        out_shape=[
            jax.ShapeDtypeStruct((_B, _K), jnp.float32),
            jax.ShapeDtypeStruct((_B, _K), jnp.float32),
        ],
        scratch_shapes=[
            pltpu.VMEM((_K, _B), jnp.float32),
            pltpu.VMEM((_K, 128), jnp.float32),
        ],
        compiler_params=pltpu.CompilerParams(
            dimension_semantics=("arbitrary", "arbitrary"),
            vmem_limit_bytes=50 * 1024 * 1024,
        ),
        interpret=interpret,
    )(z, cluster_centers)
    return (q, p)


# speedup vs baseline: 4.4266x; 3.0221x over previous
import functools

import jax
import jax.numpy as jnp
from jax.experimental import pallas as pl
from jax.experimental.pallas import tpu as pltpu

_B = 262144
_D = 128
_K = 10
_BB = 4096
_NB = _B // _BB


def _kernel(z_ref, c_ref, qt_out_ref, pt_out_ref, qt_ref, acc_ref):
    c = pl.program_id(0)
    i = pl.program_id(1)
    off = pl.multiple_of(i * _BB, _BB)

    @pl.when(c == 0)
    def _phase_a():
        zb = z_ref[...]
        cc = c_ref[...]
        zc = jax.lax.dot_general(
            cc, zb, (((1,), (1,)), ((), ())),
            preferred_element_type=jnp.float32)
        zsq = zb * zb
        ones = jnp.ones((1, _D), dtype=jnp.float32)
        zn = jax.lax.dot_general(
            ones, zsq, (((1,), (1,)), ((), ())),
            preferred_element_type=jnp.float32)
        cn = jnp.sum(cc * cc, axis=1, keepdims=True)
        sq = zn + (cn - 2.0 * zc)
        qu = 1.0 / (1.0 + sq)
        rs = jnp.sum(qu, axis=0, keepdims=True)
        qn = qu * (1.0 / rs)
        qt_ref[:, pl.ds(off, _BB)] = qn
        qt_out_ref[...] = qn
        part = qn[:, 0:128]
        for k in range(1, _BB // 128):
            part = part + qn[:, k * 128:(k + 1) * 128]

        @pl.when(i == 0)
        def _():
            acc_ref[...] = part

        @pl.when(i > 0)
        def _():
            acc_ref[...] = acc_ref[...] + part

    @pl.when(c == 1)
    def _phase_b():
        qt = qt_ref[:, pl.ds(off, _BB)]
        s = jnp.sum(acc_ref[...], axis=1, keepdims=True)
        w = (qt * qt) * (1.0 / s)
        rs = jnp.sum(w, axis=0, keepdims=True)
        pt = w * (1.0 / rs)
        pt_out_ref[...] = pt


@functools.partial(jax.jit, static_argnames=("interpret",))
def kernel(z, cluster_centers, interpret=False):
    qt, pt = pl.pallas_call(
        _kernel,
        grid=(2, _NB),
        in_specs=[
            pl.BlockSpec((_BB, _D), lambda c, i: (jnp.where(c == 0, i, _NB - 1), 0)),
            pl.BlockSpec((_K, _D), lambda c, i: (0, 0)),
        ],
        out_specs=[
            pl.BlockSpec((_K, _BB), lambda c, i: (0, jnp.where(c == 0, i, _NB - 1))),
            pl.BlockSpec((_K, _BB), lambda c, i: (0, jnp.where(c == 0, 0, i))),
        ],
        out_shape=[
            jax.ShapeDtypeStruct((_K, _B), jnp.float32),
            jax.ShapeDtypeStruct((_K, _B), jnp.float32),
        ],
        scratch_shapes=[
            pltpu.VMEM((_K, _B), jnp.float32),
            pltpu.VMEM((_K, 128), jnp.float32),
        ],
        compiler_params=pltpu.CompilerParams(
            dimension_semantics=("arbitrary", "arbitrary"),
            vmem_limit_bytes=50 * 1024 * 1024,
        ),
        interpret=interpret,
    )(z, cluster_centers)
    return (qt.T, pt.T)


# speedup vs baseline: 6.1010x; 1.3783x over previous
import functools

import jax
import jax.numpy as jnp
from jax.experimental import pallas as pl
from jax.experimental.pallas import tpu as pltpu

_B = 262144
_D = 128
_K = 10
_BB = 8192
_NB = _B // _BB


def _kernel(z_ref, c_ref, qt_out_ref, pt_out_ref, qt_ref, acc_ref):
    c = pl.program_id(0)
    i = pl.program_id(1)
    off = pl.multiple_of(i * _BB, _BB)

    @pl.when(c == 0)
    def _phase_a():
        zb = z_ref[...]
        cc = c_ref[...]
        zc = jax.lax.dot_general(
            cc, zb, (((1,), (1,)), ((), ())),
            preferred_element_type=jnp.float32)
        zsq = zb * zb
        ones = jnp.ones((1, _D), dtype=jnp.float32)
        zn = jax.lax.dot_general(
            ones, zsq, (((1,), (1,)), ((), ())),
            preferred_element_type=jnp.float32)
        cn = jnp.sum(cc * cc, axis=1, keepdims=True)
        sq = zn + (cn - 2.0 * zc)
        qu = 1.0 / (1.0 + sq)
        rs = jnp.sum(qu, axis=0, keepdims=True)
        qn = qu * (1.0 / rs)
        qt_ref[:, pl.ds(off, _BB)] = qn
        qt_out_ref[...] = qn
        part = qn[:, 0:128]
        for k in range(1, _BB // 128):
            part = part + qn[:, k * 128:(k + 1) * 128]

        @pl.when(i == 0)
        def _():
            acc_ref[...] = part

        @pl.when(i > 0)
        def _():
            acc_ref[...] = acc_ref[...] + part

    @pl.when(c == 1)
    def _phase_b():
        qt = qt_ref[:, pl.ds(off, _BB)]
        s = jnp.sum(acc_ref[...], axis=1, keepdims=True)
        w = (qt * qt) * (1.0 / s)
        rs = jnp.sum(w, axis=0, keepdims=True)
        pt = w * (1.0 / rs)
        pt_out_ref[...] = pt


@functools.partial(jax.jit, static_argnames=("interpret",))
def kernel(z, cluster_centers, interpret=False):
    qt, pt = pl.pallas_call(
        _kernel,
        grid=(2, _NB),
        in_specs=[
            pl.BlockSpec((_BB, _D), lambda c, i: (jnp.where(c == 0, i, _NB - 1), 0)),
            pl.BlockSpec((_K, _D), lambda c, i: (0, 0)),
        ],
        out_specs=[
            pl.BlockSpec((_K, _BB), lambda c, i: (0, jnp.where(c == 0, i, _NB - 1))),
            pl.BlockSpec((_K, _BB), lambda c, i: (0, jnp.where(c == 0, 0, i))),
        ],
        out_shape=[
            jax.ShapeDtypeStruct((_K, _B), jnp.float32),
            jax.ShapeDtypeStruct((_K, _B), jnp.float32),
        ],
        scratch_shapes=[
            pltpu.VMEM((_K, _B), jnp.float32),
            pltpu.VMEM((_K, 128), jnp.float32),
        ],
        compiler_params=pltpu.CompilerParams(
            dimension_semantics=("arbitrary", "arbitrary"),
            vmem_limit_bytes=50 * 1024 * 1024,
        ),
        interpret=interpret,
    )(z, cluster_centers)
    return (qt.T, pt.T)


# speedup vs baseline: 7.3904x; 1.2113x over previous
import functools

import jax
import jax.numpy as jnp
from jax.experimental import pallas as pl
from jax.experimental.pallas import tpu as pltpu

_B = 262144
_D = 128
_K = 10
_BB = 16384
_NB = _B // _BB


def _kernel(z_ref, c_ref, qt_out_ref, pt_out_ref, qt_ref, acc_ref):
    c = pl.program_id(0)
    i = pl.program_id(1)
    off = pl.multiple_of(i * _BB, _BB)

    @pl.when(c == 0)
    def _phase_a():
        zb = z_ref[...]
        cc = c_ref[...]
        zc = jax.lax.dot_general(
            cc, zb, (((1,), (1,)), ((), ())),
            preferred_element_type=jnp.float32)
        zsq = zb * zb
        ones = jnp.ones((1, _D), dtype=jnp.float32)
        zn = jax.lax.dot_general(
            ones, zsq, (((1,), (1,)), ((), ())),
            preferred_element_type=jnp.float32)
        cn = jnp.sum(cc * cc, axis=1, keepdims=True)
        sq = zn + (cn - 2.0 * zc)
        qu = 1.0 / (1.0 + sq)
        rs = jnp.sum(qu, axis=0, keepdims=True)
        qn = qu * (1.0 / rs)
        qt_ref[:, pl.ds(off, _BB)] = qn
        qt_out_ref[...] = qn
        part = qn[:, 0:128]
        for k in range(1, _BB // 128):
            part = part + qn[:, k * 128:(k + 1) * 128]

        @pl.when(i == 0)
        def _():
            acc_ref[...] = part

        @pl.when(i > 0)
        def _():
            acc_ref[...] = acc_ref[...] + part

    @pl.when(c == 1)
    def _phase_b():
        qt = qt_ref[:, pl.ds(off, _BB)]
        s = jnp.sum(acc_ref[...], axis=1, keepdims=True)
        w = (qt * qt) * (1.0 / s)
        rs = jnp.sum(w, axis=0, keepdims=True)
        pt = w * (1.0 / rs)
        pt_out_ref[...] = pt


@functools.partial(jax.jit, static_argnames=("interpret",))
def kernel(z, cluster_centers, interpret=False):
    qt, pt = pl.pallas_call(
        _kernel,
        grid=(2, _NB),
        in_specs=[
            pl.BlockSpec((_BB, _D), lambda c, i: (jnp.where(c == 0, i, _NB - 1), 0)),
            pl.BlockSpec((_K, _D), lambda c, i: (0, 0)),
        ],
        out_specs=[
            pl.BlockSpec((_K, _BB), lambda c, i: (0, jnp.where(c == 0, i, _NB - 1))),
            pl.BlockSpec((_K, _BB), lambda c, i: (0, jnp.where(c == 0, 0, i))),
        ],
        out_shape=[
            jax.ShapeDtypeStruct((_K, _B), jnp.float32),
            jax.ShapeDtypeStruct((_K, _B), jnp.float32),
        ],
        scratch_shapes=[
            pltpu.VMEM((_K, _B), jnp.float32),
            pltpu.VMEM((_K, 128), jnp.float32),
        ],
        compiler_params=pltpu.CompilerParams(
            dimension_semantics=("arbitrary", "arbitrary"),
            vmem_limit_bytes=50 * 1024 * 1024,
        ),
        interpret=interpret,
    )(z, cluster_centers)
    return (qt.T, pt.T)
